# baseline (speedup 1.0000x reference)
import functools

import jax
import jax.numpy as jnp
from jax import lax
from jax.experimental import pallas as pl
from jax.experimental.pallas import tpu as pltpu
from jax.experimental.pallas import tpu_sc as plsc

_INFO = plsc.get_sparse_core_info()
_NC, _NS = _INFO.num_cores, _INFO.num_subcores
_NW = _NC * _NS

_L = 48
_D = 1024
_ROWS_PER_W = 8
_ACTIVE_W = _L // _ROWS_PER_W


@functools.partial(
    pl.kernel,
    out_type=jax.ShapeDtypeStruct((_L, _D), jnp.float32),
    mesh=plsc.VectorSubcoreMesh(core_axis_name="c", subcore_axis_name="s"),
    scratch_types=[
        pltpu.VMEM((_ROWS_PER_W,), jnp.int32),
        pltpu.VMEM((_ROWS_PER_W, _D), jnp.float32),
        pltpu.SemaphoreType.DMA,
    ],
)
def _sc_gather(pe_hbm, idx_hbm, sel_hbm, idx_v, rows_v, sem):
    wid = lax.axis_index("s") * _NC + lax.axis_index("c")

    @pl.when(wid < _ACTIVE_W)
    def _():
        base = wid * _ROWS_PER_W
        pltpu.sync_copy(idx_hbm.at[pl.ds(base, _ROWS_PER_W)], idx_v)
        pltpu.async_copy(pe_hbm.at[idx_v], rows_v, sem).wait()
        pltpu.sync_copy(rows_v, sel_hbm.at[pl.ds(base, _ROWS_PER_W), :])


def _add_body(sel_ref, x_ref, o_ref):
    o_ref[...] = x_ref[...] + sel_ref[...]


def _tc_add(x2, sel2, b_blk):
    b = x2.shape[0]
    w = x2.shape[1]
    return pl.pallas_call(
        _add_body,
        grid=(b // b_blk,),
        in_specs=[
            pl.BlockSpec((1, w), lambda i: (0, 0)),
            pl.BlockSpec((b_blk, w), lambda i: (i, 0)),
        ],
        out_specs=pl.BlockSpec((b_blk, w), lambda i: (i, 0)),
        out_shape=jax.ShapeDtypeStruct((b, w), jnp.float32),
        compiler_params=pltpu.CompilerParams(
            dimension_semantics=("arbitrary",),
        ),
    )(sel2, x2)


def kernel(x, pe, layer_indices):
    sel = _sc_gather(pe, layer_indices.astype(jnp.int32))
    b, l, d = x.shape
    out2 = _tc_add(x.reshape(b, l * d), sel.reshape(1, l * d), 16)
    return out2.reshape(b, l, d)


# speedup vs baseline: 2.8826x; 2.8826x over previous
import functools

import jax
import jax.numpy as jnp
from jax import lax
from jax.experimental import pallas as pl
from jax.experimental.pallas import tpu as pltpu
from jax.experimental.pallas import tpu_sc as plsc

_INFO = plsc.get_sparse_core_info()
_NC, _NS = _INFO.num_cores, _INFO.num_subcores
_NW = _NC * _NS

_L = 48
_D = 1024
_ROWS_PER_W = 8
_ACTIVE_W = _L // _ROWS_PER_W


@functools.partial(
    pl.kernel,
    out_type=jax.ShapeDtypeStruct((_L, _D), jnp.float32),
    mesh=plsc.VectorSubcoreMesh(core_axis_name="c", subcore_axis_name="s"),
    scratch_types=[
        pltpu.VMEM((_ROWS_PER_W,), jnp.int32),
        pltpu.VMEM((_ROWS_PER_W, _D), jnp.float32),
        pltpu.SemaphoreType.DMA,
    ],
)
def _sc_gather(pe_hbm, idx_hbm, sel_hbm, idx_v, rows_v, sem):
    wid = lax.axis_index("s") * _NC + lax.axis_index("c")

    @pl.when(wid < _ACTIVE_W)
    def _():
        base = wid * _ROWS_PER_W
        pltpu.sync_copy(idx_hbm.at[pl.ds(base, _ROWS_PER_W)], idx_v)
        pltpu.async_copy(pe_hbm.at[idx_v], rows_v, sem).wait()
        pltpu.sync_copy(rows_v, sel_hbm.at[pl.ds(base, _ROWS_PER_W), :])


def _add_body(sel_ref, x_ref, o_ref):
    o_ref[...] = x_ref[...] + sel_ref[...][None]


def _tc_add(x, sel, b_blk):
    b, l, d = x.shape
    return pl.pallas_call(
        _add_body,
        grid=(b // b_blk,),
        in_specs=[
            pl.BlockSpec((l, d), lambda i: (0, 0)),
            pl.BlockSpec((b_blk, l, d), lambda i: (i, 0, 0)),
        ],
        out_specs=pl.BlockSpec((b_blk, l, d), lambda i: (i, 0, 0)),
        out_shape=jax.ShapeDtypeStruct((b, l, d), jnp.float32),
        compiler_params=pltpu.CompilerParams(
            dimension_semantics=("arbitrary",),
        ),
    )(sel, x)


def kernel(x, pe, layer_indices):
    sel = _sc_gather(pe, layer_indices.astype(jnp.int32))
    return _tc_add(x, sel, 16)


# speedup vs baseline: 2.9597x; 1.0267x over previous
import functools

import jax
import jax.numpy as jnp
from jax import lax
from jax.experimental import pallas as pl
from jax.experimental.pallas import tpu as pltpu
from jax.experimental.pallas import tpu_sc as plsc

_INFO = plsc.get_sparse_core_info()
_NC, _NS = _INFO.num_cores, _INFO.num_subcores
_NW = _NC * _NS

_L = 48
_D = 1024
_ROWS_PER_W = 8
_ACTIVE_W = _L // _ROWS_PER_W


@functools.partial(
    pl.kernel,
    out_type=jax.ShapeDtypeStruct((_L, _D), jnp.float32),
    mesh=plsc.VectorSubcoreMesh(core_axis_name="c", subcore_axis_name="s"),
    scratch_types=[
        pltpu.VMEM((_ROWS_PER_W,), jnp.int32),
        pltpu.VMEM((_ROWS_PER_W, _D), jnp.float32),
        pltpu.SemaphoreType.DMA,
    ],
)
def _sc_gather(pe_hbm, idx_hbm, sel_hbm, idx_v, rows_v, sem):
    wid = lax.axis_index("s") * _NC + lax.axis_index("c")

    @pl.when(wid < _ACTIVE_W)
    def _():
        base = wid * _ROWS_PER_W
        pltpu.sync_copy(idx_hbm.at[pl.ds(base, _ROWS_PER_W)], idx_v)
        pltpu.async_copy(pe_hbm.at[idx_v], rows_v, sem).wait()
        pltpu.sync_copy(rows_v, sel_hbm.at[pl.ds(base, _ROWS_PER_W), :])


def _add_body(sel_ref, x_ref, o_ref):
    o_ref[...] = x_ref[...] + sel_ref[...][None]


def _tc_add(x, sel, b_blk):
    b, l, d = x.shape
    return pl.pallas_call(
        _add_body,
        grid=(b // b_blk,),
        in_specs=[
            pl.BlockSpec((l, d), lambda i: (0, 0)),
            pl.BlockSpec((b_blk, l, d), lambda i: (i, 0, 0)),
        ],
        out_specs=pl.BlockSpec((b_blk, l, d), lambda i: (i, 0, 0)),
        out_shape=jax.ShapeDtypeStruct((b, l, d), jnp.float32),
        compiler_params=pltpu.CompilerParams(
            dimension_semantics=("arbitrary",),
        ),
    )(sel, x)


def kernel(x, pe, layer_indices):
    sel = _sc_gather(pe, layer_indices.astype(jnp.int32))
    return _tc_add(x, sel, 32)


# speedup vs baseline: 2.9781x; 1.0062x over previous
import functools

import jax
import jax.numpy as jnp
from jax import lax
from jax.experimental import pallas as pl
from jax.experimental.pallas import tpu as pltpu
from jax.experimental.pallas import tpu_sc as plsc

_INFO = plsc.get_sparse_core_info()
_NC, _NS = _INFO.num_cores, _INFO.num_subcores
_NW = _NC * _NS

_L = 48
_D = 1024
_ROWS_PER_W = 8
_ACTIVE_W = _L // _ROWS_PER_W


@functools.partial(
    pl.kernel,
    out_type=jax.ShapeDtypeStruct((_L, _D), jnp.float32),
    mesh=plsc.VectorSubcoreMesh(core_axis_name="c", subcore_axis_name="s"),
    scratch_types=[
        pltpu.VMEM((_ROWS_PER_W,), jnp.int32),
        pltpu.VMEM((_ROWS_PER_W, _D), jnp.float32),
        pltpu.SemaphoreType.DMA,
    ],
)
def _sc_gather(pe_hbm, idx_hbm, sel_hbm, idx_v, rows_v, sem):
    wid = lax.axis_index("s") * _NC + lax.axis_index("c")

    @pl.when(wid < _ACTIVE_W)
    def _():
        base = wid * _ROWS_PER_W
        pltpu.sync_copy(idx_hbm.at[pl.ds(base, _ROWS_PER_W)], idx_v)
        pltpu.async_copy(pe_hbm.at[idx_v], rows_v, sem).wait()
        pltpu.sync_copy(rows_v, sel_hbm.at[pl.ds(base, _ROWS_PER_W), :])


def _add_body(sel_ref, x_ref, o_ref):
    o_ref[...] = x_ref[...] + sel_ref[...][None]


def _tc_add(x, sel, b_blk):
    b, l, d = x.shape
    return pl.pallas_call(
        _add_body,
        grid=(b // b_blk,),
        in_specs=[
            pl.BlockSpec((l, d), lambda i: (0, 0)),
            pl.BlockSpec((b_blk, l, d), lambda i: (i, 0, 0)),
        ],
        out_specs=pl.BlockSpec((b_blk, l, d), lambda i: (i, 0, 0)),
        out_shape=jax.ShapeDtypeStruct((b, l, d), jnp.float32),
        compiler_params=pltpu.CompilerParams(
            dimension_semantics=("arbitrary",),
        ),
    )(sel, x)


def kernel(x, pe, layer_indices):
    sel = _sc_gather(pe, layer_indices.astype(jnp.int32))
    return _tc_add(x, sel, 64)


# speedup vs baseline: 3.0096x; 1.0106x over previous
import functools

import jax
import jax.numpy as jnp
from jax import lax
from jax.experimental import pallas as pl
from jax.experimental.pallas import tpu as pltpu
from jax.experimental.pallas import tpu_sc as plsc

_INFO = plsc.get_sparse_core_info()
_NC, _NS = _INFO.num_cores, _INFO.num_subcores
_NW = _NC * _NS

_L = 48
_D = 1024
_ROWS_PER_W = 8
_ACTIVE_W = _L // _ROWS_PER_W


@functools.partial(
    pl.kernel,
    out_type=jax.ShapeDtypeStruct((_L, _D), jnp.float32),
    mesh=plsc.VectorSubcoreMesh(core_axis_name="c", subcore_axis_name="s", num_cores=1),
    scratch_types=[
        pltpu.VMEM((_ROWS_PER_W,), jnp.int32),
        pltpu.VMEM((_ROWS_PER_W, _D), jnp.float32),
        pltpu.SemaphoreType.DMA,
    ],
)
def _sc_gather(pe_hbm, idx_hbm, sel_hbm, idx_v, rows_v, sem):
    wid = lax.axis_index("s") * _NC + lax.axis_index("c")

    @pl.when(wid < _ACTIVE_W)
    def _():
        base = wid * _ROWS_PER_W
        pltpu.sync_copy(idx_hbm.at[pl.ds(base, _ROWS_PER_W)], idx_v)
        pltpu.async_copy(pe_hbm.at[idx_v], rows_v, sem).wait()
        pltpu.sync_copy(rows_v, sel_hbm.at[pl.ds(base, _ROWS_PER_W), :])


def _add_body(sel_ref, x_ref, o_ref):
    o_ref[...] = x_ref[...] + sel_ref[...][None]


def _tc_add(x, sel, b_blk):
    b, l, d = x.shape
    return pl.pallas_call(
        _add_body,
        grid=(b // b_blk,),
        in_specs=[
            pl.BlockSpec((l, d), lambda i: (0, 0)),
            pl.BlockSpec((b_blk, l, d), lambda i: (i, 0, 0)),
        ],
        out_specs=pl.BlockSpec((b_blk, l, d), lambda i: (i, 0, 0)),
        out_shape=jax.ShapeDtypeStruct((b, l, d), jnp.float32),
        compiler_params=pltpu.CompilerParams(
            dimension_semantics=("arbitrary",),
        ),
    )(sel, x)


def kernel(x, pe, layer_indices):
    sel = _sc_gather(pe, layer_indices.astype(jnp.int32))
    return _tc_add(x, sel, 64)
